# baseline (speedup 1.0000x reference)
import functools

import jax
import jax.numpy as jnp
from jax import lax
from jax.experimental import pallas as pl
from jax.experimental.pallas import tpu as pltpu
from jax.experimental.pallas import tpu_sc as plsc

N = 10000
NP = 10240
E = 320000
D = 128
DH = 256
NC = 2
NS = 16
NW = NC * NS
EPW = E // NW
CH = 80
NG = 5
NIS = 26
NI = NG * NIS
EPWP = NI * CH
RPT = NP // NS


def _sc_aggregate(x, src4, dst4, zrows, orows):
    mesh = plsc.VectorSubcoreMesh(core_axis_name="c", subcore_axis_name="s")

    @functools.partial(
        pl.kernel,
        out_type=[
            jax.ShapeDtypeStruct((NC * NP, D), jnp.float32),
            jax.ShapeDtypeStruct((NC * NP, D), jnp.float32),
        ],
        mesh=mesh,
        scratch_types=[
            pltpu.VMEM((NIS, CH), jnp.int32),
            pltpu.VMEM((NIS, CH), jnp.int32),
            pltpu.VMEM((CH, D), jnp.float32),
            pltpu.VMEM((CH, D), jnp.float32),
            pltpu.VMEM_SHARED((NP, D), jnp.float32),
            pltpu.SemaphoreType.DMA,
            pltpu.SemaphoreType.DMA,
        ],
    )
    def k(x_hbm, src_hbm, dst_hbm, zr_hbm, on_hbm, agg_out, deg_out,
          src_v, dst_v, rows0, rows1, acc_sh, sem0, sem1):
        c = lax.axis_index("c")
        s = lax.axis_index("s")
        wid = c * NS + s
        tb = s * RPT

        pltpu.sync_copy(zr_hbm.at[pl.ds(tb, RPT)], acc_sh.at[pl.ds(tb, RPT)])
        plsc.subcore_barrier()

        def outer_a(g, carry):
            pltpu.sync_copy(src_hbm.at[wid, g], src_v)
            pltpu.sync_copy(dst_hbm.at[wid, g], dst_v)

            def body(j, c2):
                h0 = pltpu.async_copy(
                    x_hbm.at[src_v.at[2 * j]], rows0, sem0)
                h1 = pltpu.async_copy(
                    x_hbm.at[src_v.at[2 * j + 1]], rows1, sem1)
                h0.wait()
                pltpu.sync_copy(rows0, acc_sh.at[dst_v.at[2 * j]], add=True)
                h1.wait()
                pltpu.sync_copy(
                    rows1, acc_sh.at[dst_v.at[2 * j + 1]], add=True)
                return c2

            lax.fori_loop(0, NIS // 2, body, 0)
            return carry

        lax.fori_loop(0, NG, outer_a, 0)

        plsc.subcore_barrier()
        pltpu.sync_copy(acc_sh.at[pl.ds(tb, RPT)],
                        agg_out.at[pl.ds(c * NP + tb, RPT)])
        plsc.subcore_barrier()

        pltpu.sync_copy(zr_hbm.at[pl.ds(tb, RPT)], acc_sh.at[pl.ds(tb, RPT)])
        pltpu.sync_copy(on_hbm, rows0)
        plsc.subcore_barrier()

        def outer_b(g, carry):
            pltpu.sync_copy(dst_hbm.at[wid, g], dst_v)

            def body(i, c2):
                pltpu.sync_copy(rows0, acc_sh.at[dst_v.at[i]], add=True)
                return c2

            lax.fori_loop(0, NIS, body, 0)
            return carry

        lax.fori_loop(0, NG, outer_b, 0)

        plsc.subcore_barrier()
        pltpu.sync_copy(acc_sh.at[pl.ds(tb, RPT)],
                        deg_out.at[pl.ds(c * NP + tb, RPT)])

    return k(x, src4, dst4, zrows, orows)


def _tc_combine(aggp, degp, x, w_cat, b):
    R = 1024
    grid = (NP // R,)

    def body(aggp_ref, degp_ref, x_ref, w_ref, b_ref, o_ref):
        a = aggp_ref[0] + aggp_ref[1]
        d = degp_ref[0][:, 0:1] + degp_ref[1][:, 0:1]
        mean = a / jnp.maximum(d, 1.0)
        cat = jnp.concatenate([mean, x_ref[...]], axis=1)
        o_ref[...] = (
            jnp.dot(cat, w_ref[...], preferred_element_type=jnp.float32)
            + b_ref[...]
        )

    return pl.pallas_call(
        body,
        grid=grid,
        in_specs=[
            pl.BlockSpec((2, R, D), lambda i: (0, i, 0)),
            pl.BlockSpec((2, R, D), lambda i: (0, i, 0)),
            pl.BlockSpec((R, D), lambda i: (i, 0)),
            pl.BlockSpec((2 * D, DH), lambda i: (0, 0)),
            pl.BlockSpec((1, DH), lambda i: (0, 0)),
        ],
        out_specs=pl.BlockSpec((R, DH), lambda i: (i, 0)),
        out_shape=jax.ShapeDtypeStruct((NP, DH), jnp.float32),
    )(aggp, degp, x, w_cat, b)


def kernel(x, msg_pass_edge_index, W_l, b_l, W_r):
    pad = EPWP - EPW
    src2 = msg_pass_edge_index[0].reshape(NW, EPW)
    dst2 = msg_pass_edge_index[1].reshape(NW, EPW)
    pad_src = jnp.zeros((NW, pad), jnp.int32)
    pad_dst = jnp.broadcast_to(
        N + jnp.arange(pad, dtype=jnp.int32) % (NP - N), (NW, pad))
    src4 = jnp.concatenate([src2, pad_src], axis=1).reshape(NW, NG, NIS, CH)
    dst4 = jnp.concatenate([dst2, pad_dst], axis=1).reshape(NW, NG, NIS, CH)
    zrows = jnp.zeros((NP, D), jnp.float32)
    orows = jnp.ones((CH, D), jnp.float32)
    x_pad = jnp.pad(x, ((0, NP - N), (0, 0)))
    aggp, degp = _sc_aggregate(x_pad, src4, dst4, zrows, orows)
    w_cat = jnp.concatenate([W_l.T, W_r.T], axis=0)
    b = b_l.reshape(1, DH)
    out = _tc_combine(
        aggp.reshape(NC, NP, D), degp.reshape(NC, NP, D), x_pad, w_cat, b
    )
    return out[:N]


# speedup vs baseline: 2.5903x; 2.5903x over previous
import functools

import jax
import jax.numpy as jnp
from jax import lax
from jax.experimental import pallas as pl
from jax.experimental.pallas import tpu as pltpu
from jax.experimental.pallas import tpu_sc as plsc

N = 10000
NP = 10240
E = 320000
D = 128
DH = 256
NC = 2
NS = 16
NW = NC * NS
EPW = E // NW
CH = 80
NG = 5
NIS = 25
NI = NG * NIS
RPT = NP // NS


def _sc_aggregate(x, src4, dst4, zrows, orows):
    mesh = plsc.VectorSubcoreMesh(core_axis_name="c", subcore_axis_name="s")

    @functools.partial(
        pl.kernel,
        out_type=[
            jax.ShapeDtypeStruct((NC * NP, D), jnp.float32),
            jax.ShapeDtypeStruct((NC * NP, D), jnp.float32),
        ],
        mesh=mesh,
        scratch_types=[
            pltpu.VMEM((NIS, CH), jnp.int32),
            pltpu.VMEM((NIS, CH), jnp.int32),
            pltpu.VMEM((CH, D), jnp.float32),
            pltpu.VMEM((CH, D), jnp.float32),
            pltpu.VMEM_SHARED((NP, D), jnp.float32),
            pltpu.SemaphoreType.DMA,
            pltpu.SemaphoreType.DMA,
        ],
    )
    def k(x_hbm, src_hbm, dst_hbm, zr_hbm, on_hbm, agg_out, deg_out,
          src_v, dst_v, rows0, rows1, acc_sh, sem0, sem1):
        c = lax.axis_index("c")
        s = lax.axis_index("s")
        wid = c * NS + s
        tb = s * RPT

        pltpu.sync_copy(zr_hbm.at[pl.ds(tb, RPT)], acc_sh.at[pl.ds(tb, RPT)])
        plsc.subcore_barrier()

        def outer_a(g, carry):
            pltpu.sync_copy(src_hbm.at[wid, g], src_v)
            pltpu.sync_copy(dst_hbm.at[wid, g], dst_v)

            def body(j, c2):
                h0 = pltpu.async_copy(
                    x_hbm.at[src_v.at[2 * j]], rows0, sem0)
                h1 = pltpu.async_copy(
                    x_hbm.at[src_v.at[2 * j + 1]], rows1, sem1)
                h0.wait()
                pltpu.sync_copy(rows0, acc_sh.at[dst_v.at[2 * j]], add=True)
                h1.wait()
                pltpu.sync_copy(
                    rows1, acc_sh.at[dst_v.at[2 * j + 1]], add=True)
                return c2

            lax.fori_loop(0, NIS // 2, body, 0)
            pltpu.async_copy(
                x_hbm.at[src_v.at[NIS - 1]], rows0, sem0).wait()
            pltpu.sync_copy(rows0, acc_sh.at[dst_v.at[NIS - 1]], add=True)
            return carry

        lax.fori_loop(0, NG, outer_a, 0)

        plsc.subcore_barrier()
        pltpu.sync_copy(acc_sh.at[pl.ds(tb, RPT)],
                        agg_out.at[pl.ds(c * NP + tb, RPT)])
        plsc.subcore_barrier()

        pltpu.sync_copy(zr_hbm.at[pl.ds(tb, RPT)], acc_sh.at[pl.ds(tb, RPT)])
        pltpu.sync_copy(on_hbm, rows0)
        plsc.subcore_barrier()

        def outer_b(g, carry):
            pltpu.sync_copy(dst_hbm.at[wid, g], dst_v)

            def body(i, c2):
                pltpu.sync_copy(rows0, acc_sh.at[dst_v.at[i]], add=True)
                return c2

            lax.fori_loop(0, NIS, body, 0)
            return carry

        lax.fori_loop(0, NG, outer_b, 0)

        plsc.subcore_barrier()
        pltpu.sync_copy(acc_sh.at[pl.ds(tb, RPT)],
                        deg_out.at[pl.ds(c * NP + tb, RPT)])

    return k(x, src4, dst4, zrows, orows)


def _tc_combine(aggp, degp, x, w_cat, b):
    R = 1024
    grid = (NP // R,)

    def body(aggp_ref, degp_ref, x_ref, w_ref, b_ref, o_ref):
        a = aggp_ref[0] + aggp_ref[1]
        d = degp_ref[0][:, 0:1] + degp_ref[1][:, 0:1]
        mean = a / jnp.maximum(d, 1.0)
        cat = jnp.concatenate([mean, x_ref[...]], axis=1)
        o_ref[...] = (
            jnp.dot(cat, w_ref[...], preferred_element_type=jnp.float32)
            + b_ref[...]
        )

    return pl.pallas_call(
        body,
        grid=grid,
        in_specs=[
            pl.BlockSpec((2, R, D), lambda i: (0, i, 0)),
            pl.BlockSpec((2, R, D), lambda i: (0, i, 0)),
            pl.BlockSpec((R, D), lambda i: (i, 0)),
            pl.BlockSpec((2 * D, DH), lambda i: (0, 0)),
            pl.BlockSpec((1, DH), lambda i: (0, 0)),
        ],
        out_specs=pl.BlockSpec((R, DH), lambda i: (i, 0)),
        out_shape=jax.ShapeDtypeStruct((NP, DH), jnp.float32),
    )(aggp, degp, x, w_cat, b)


def kernel(x, msg_pass_edge_index, W_l, b_l, W_r):
    src4 = msg_pass_edge_index[0].reshape(NW, NG, NIS, CH)
    dst4 = msg_pass_edge_index[1].reshape(NW, NG, NIS, CH)
    zrows = jnp.zeros((NP, D), jnp.float32)
    orows = jnp.ones((CH, D), jnp.float32)
    x_pad = jnp.pad(x, ((0, NP - N), (0, 0)))
    aggp, degp = _sc_aggregate(x_pad, src4, dst4, zrows, orows)
    w_cat = jnp.concatenate([W_l.T, W_r.T], axis=0)
    b = b_l.reshape(1, DH)
    out = _tc_combine(
        aggp.reshape(NC, NP, D), degp.reshape(NC, NP, D), x_pad, w_cat, b
    )
    return out[:N]


# speedup vs baseline: 2.5926x; 1.0009x over previous
import functools

import jax
import jax.numpy as jnp
from jax import lax
from jax.experimental import pallas as pl
from jax.experimental.pallas import tpu as pltpu
from jax.experimental.pallas import tpu_sc as plsc

N = 10000
NP = 10240
E = 320000
D = 128
DH = 256
NC = 2
NS = 16
NW = NC * NS
EPW = E // NW
CH = 80
NG = 5
NIS = 25
NI = NG * NIS
RPT = NP // NS


def _sc_aggregate(x, src4, dst4, zrows, orows):
    mesh = plsc.VectorSubcoreMesh(core_axis_name="c", subcore_axis_name="s")

    @functools.partial(
        pl.kernel,
        out_type=[
            jax.ShapeDtypeStruct((NC * NP, D), jnp.float32),
            jax.ShapeDtypeStruct((NC * NP, D), jnp.float32),
        ],
        mesh=mesh,
        scratch_types=[
            pltpu.VMEM((NIS, CH), jnp.int32),
            pltpu.VMEM((NIS, CH), jnp.int32),
            pltpu.VMEM((CH, D), jnp.float32),
            pltpu.VMEM((CH, D), jnp.float32),
            pltpu.VMEM_SHARED((NP, D), jnp.float32),
            pltpu.SemaphoreType.DMA,
            pltpu.SemaphoreType.DMA,
        ],
    )
    def k(x_hbm, src_hbm, dst_hbm, zr_hbm, on_hbm, agg_out, deg_out,
          src_v, dst_v, rows0, rows1, acc_sh, sem0, sem1):
        c = lax.axis_index("c")
        s = lax.axis_index("s")
        wid = c * NS + s
        tb = s * RPT

        pltpu.sync_copy(zr_hbm.at[pl.ds(tb, RPT)], acc_sh.at[pl.ds(tb, RPT)])
        plsc.subcore_barrier()

        def outer_a(g, carry):
            pltpu.sync_copy(src_hbm.at[wid, g], src_v)
            pltpu.sync_copy(dst_hbm.at[wid, g], dst_v)

            def body(j, c2):
                h0 = pltpu.async_copy(
                    x_hbm.at[src_v.at[2 * j]], rows0, sem0)
                h1 = pltpu.async_copy(
                    x_hbm.at[src_v.at[2 * j + 1]], rows1, sem1)
                h0.wait()
                pltpu.sync_copy(rows0, acc_sh.at[dst_v.at[2 * j]], add=True)
                h1.wait()
                pltpu.sync_copy(
                    rows1, acc_sh.at[dst_v.at[2 * j + 1]], add=True)
                return c2

            lax.fori_loop(0, NIS // 2, body, 0)
            pltpu.async_copy(
                x_hbm.at[src_v.at[NIS - 1]], rows0, sem0).wait()
            pltpu.sync_copy(rows0, acc_sh.at[dst_v.at[NIS - 1]], add=True)
            return carry

        lax.fori_loop(0, NG, outer_a, 0)

        plsc.subcore_barrier()
        pltpu.sync_copy(acc_sh.at[pl.ds(tb, RPT)],
                        agg_out.at[pl.ds(c * NP + tb, RPT)])
        plsc.subcore_barrier()

        pltpu.sync_copy(zr_hbm.at[pl.ds(tb, RPT)], acc_sh.at[pl.ds(tb, RPT)])
        pltpu.sync_copy(on_hbm, rows0)
        plsc.subcore_barrier()

        def outer_b(g, carry):
            pltpu.sync_copy(dst_hbm.at[wid, g], dst_v)

            def body(j, c2):
                h0 = pltpu.async_copy(
                    rows0, acc_sh.at[dst_v.at[2 * j]], sem0, add=True)
                h1 = pltpu.async_copy(
                    rows0, acc_sh.at[dst_v.at[2 * j + 1]], sem1, add=True)
                h0.wait()
                h1.wait()
                return c2

            lax.fori_loop(0, NIS // 2, body, 0)
            pltpu.sync_copy(rows0, acc_sh.at[dst_v.at[NIS - 1]], add=True)
            return carry

        lax.fori_loop(0, NG, outer_b, 0)

        plsc.subcore_barrier()
        pltpu.sync_copy(acc_sh.at[pl.ds(tb, RPT)],
                        deg_out.at[pl.ds(c * NP + tb, RPT)])

    return k(x, src4, dst4, zrows, orows)


def _tc_combine(aggp, degp, x, w_cat, b):
    R = 1024
    grid = (NP // R,)

    def body(aggp_ref, degp_ref, x_ref, w_ref, b_ref, o_ref):
        a = aggp_ref[0] + aggp_ref[1]
        d = degp_ref[0][:, 0:1] + degp_ref[1][:, 0:1]
        mean = a / jnp.maximum(d, 1.0)
        cat = jnp.concatenate([mean, x_ref[...]], axis=1)
        o_ref[...] = (
            jnp.dot(cat, w_ref[...], preferred_element_type=jnp.float32)
            + b_ref[...]
        )

    return pl.pallas_call(
        body,
        grid=grid,
        in_specs=[
            pl.BlockSpec((2, R, D), lambda i: (0, i, 0)),
            pl.BlockSpec((2, R, D), lambda i: (0, i, 0)),
            pl.BlockSpec((R, D), lambda i: (i, 0)),
            pl.BlockSpec((2 * D, DH), lambda i: (0, 0)),
            pl.BlockSpec((1, DH), lambda i: (0, 0)),
        ],
        out_specs=pl.BlockSpec((R, DH), lambda i: (i, 0)),
        out_shape=jax.ShapeDtypeStruct((NP, DH), jnp.float32),
    )(aggp, degp, x, w_cat, b)


def kernel(x, msg_pass_edge_index, W_l, b_l, W_r):
    src4 = msg_pass_edge_index[0].reshape(NW, NG, NIS, CH)
    dst4 = msg_pass_edge_index[1].reshape(NW, NG, NIS, CH)
    zrows = jnp.zeros((NP, D), jnp.float32)
    orows = jnp.ones((CH, D), jnp.float32)
    x_pad = jnp.pad(x, ((0, NP - N), (0, 0)))
    aggp, degp = _sc_aggregate(x_pad, src4, dst4, zrows, orows)
    w_cat = jnp.concatenate([W_l.T, W_r.T], axis=0)
    b = b_l.reshape(1, DH)
    out = _tc_combine(
        aggp.reshape(NC, NP, D), degp.reshape(NC, NP, D), x_pad, w_cat, b
    )
    return out[:N]
